# Initial kernel scaffold; baseline (speedup 1.0000x reference)
import jax
import jax.numpy as jnp
from jax.experimental import pallas as pl


def kernel(x, edge_index, h, c, W_gcn, b_gcn, W_ih, W_hh, b_ih, b_hh, W_out, b_out):
    raise NotImplementedError("write your pallas kernel here")


# speedup vs baseline: 3.6756x; 3.6756x over previous
import jax
import jax.numpy as jnp
from jax import lax
from jax.experimental import pallas as pl
from jax.experimental.pallas import tpu as pltpu
from jax.experimental.pallas import tpu_sc as plsc

N_NODES = 10000
N_EDGES = 160000
NPAD = 10240
EPB = 128
TB = 80
EPAD = 16 * TB * EPB
NC = 2
NS = 16
ROWS_PER_TILE = NPAD // NS

def _mesh():
    return plsc.VectorSubcoreMesh(
        core_axis_name="c", subcore_axis_name="s", num_cores=NC, num_subcores=NS)


def _sc_deg(dstp32):
    @pl.kernel(
        out_type=jax.ShapeDtypeStruct((NC * NPAD,), jnp.float32),
        mesh=_mesh(),
        scratch_types=[
            pltpu.VMEM((TB // 2, EPB), jnp.int32),
            pltpu.VMEM((EPB,), jnp.float32),
            pltpu.VMEM((ROWS_PER_TILE,), jnp.float32),
            pltpu.VMEM_SHARED((NPAD,), jnp.float32),
        ],
    )
    def deg_kernel(dst_hbm, deg_hbm, dst_v, ones_v, z_v, deg_sp):
        c = lax.axis_index("c")
        s = lax.axis_index("s")
        gw = c * NS + s
        for i in range(EPB // 16):
            ones_v[pl.ds(i * 16, 16)] = jnp.full((16,), 1.0, jnp.float32)
        for i in range(ROWS_PER_TILE // 16):
            z_v[pl.ds(i * 16, 16)] = jnp.zeros((16,), jnp.float32)
        pltpu.sync_copy(z_v, deg_sp.at[pl.ds(s * ROWS_PER_TILE, ROWS_PER_TILE)])
        pltpu.sync_copy(dst_hbm.at[gw], dst_v)
        plsc.subcore_barrier()
        def blk(b, carry):
            pltpu.sync_copy(ones_v, deg_sp.at[dst_v.at[b]], add=True)
            return carry
        lax.fori_loop(0, TB // 2, blk, 0)
        plsc.subcore_barrier()
        pltpu.sync_copy(deg_sp.at[pl.ds(s * ROWS_PER_TILE, ROWS_PER_TILE)],
                        deg_hbm.at[pl.ds(c * NPAD + s * ROWS_PER_TILE, ROWS_PER_TILE)])

    return deg_kernel(dstp32)


def _tc_gcn_mm(x, dv2, w):
    rb = 400
    nb = N_NODES // rb

    def mm_kernel(x_ref, deg_ref, w_ref, o_ref):
        dv = lax.rsqrt(deg_ref[...] + 1.0)
        o_ref[0] = jnp.dot(x_ref[...] * dv, w_ref[...],
                           preferred_element_type=jnp.float32)

    return pl.pallas_call(
        mm_kernel,
        grid=(nb, 4),
        in_specs=[
            pl.BlockSpec((rb, x.shape[1]), lambda i, j: (i, 0)),
            pl.BlockSpec((rb, 1), lambda i, j: (i, 0)),
            pl.BlockSpec((x.shape[1], 128), lambda i, j: (0, j)),
        ],
        out_specs=pl.BlockSpec((1, rb, 128), lambda i, j: (j, i, 0)),
        out_shape=jax.ShapeDtypeStruct((4, N_NODES, 128), jnp.float32),
    )(x, dv2, w)


NHALF = NPAD // 2
ACC_ROWS = NHALF + 64
HROWS = NHALF // NS


def _sc_scatter(xs_flat, sadj, dst2, zz):
    @pl.kernel(
        out_type=jax.ShapeDtypeStruct((8 * NHALF, 128), jnp.float32),
        mesh=_mesh(),
        scratch_types=[
            pltpu.VMEM((TB, EPB), jnp.int32),
            pltpu.VMEM((TB, EPB), jnp.int32),
            pltpu.VMEM((EPB, 128), jnp.float32),
            pltpu.VMEM((HROWS, 128), jnp.float32),
            pltpu.VMEM_SHARED((ACC_ROWS, 128), jnp.float32),
            pltpu.SemaphoreType.DMA,
        ],
    )
    def msg_kernel(xs_hbm, sadj_hbm, dst_hbm, zz_hbm, acc_hbm,
                   idx_v, dst_v, g_v, z_v, acc_sp, sem):
        c = lax.axis_index("c")
        s = lax.axis_index("s")
        pltpu.sync_copy(zz_hbm, z_v)
        for t in range(4):
            chunk = c + 2 * (t // 2)
            half = t % 2
            pltpu.sync_copy(z_v, acc_sp.at[pl.ds(s * HROWS, HROWS)])
            pltpu.sync_copy(sadj_hbm.at[chunk, s], idx_v)
            pltpu.sync_copy(dst_hbm.at[half, s], dst_v)
            plsc.subcore_barrier()
            def blk(b, carry):
                pltpu.async_copy(xs_hbm.at[idx_v.at[b]], g_v, sem).wait()
                pltpu.sync_copy(g_v, acc_sp.at[dst_v.at[b]], add=True)
                return carry
            lax.fori_loop(0, TB, blk, 0)
            plsc.subcore_barrier()
            pltpu.sync_copy(
                acc_sp.at[pl.ds(s * HROWS, HROWS)],
                acc_hbm.at[pl.ds((chunk * 2 + half) * NHALF + s * HROWS, HROWS)])
            plsc.subcore_barrier()

    return msg_kernel(xs_flat, sadj, dst2, zz)


def _tc_lstm(acc4, xs4, dv2, h, c, whhT, wihT, bihh, bg, wo, bo):
    rb = 1000
    nb = N_NODES // rb
    H = h.shape[1]

    def lstm_kernel(acc_ref, xs_ref, deg_ref, h_ref, c_ref, whh_ref, wih_ref,
                    bihh_ref, bg_ref, wo_ref, bo_ref, out_ref, hn_ref, cn_ref):
        dv = lax.rsqrt(deg_ref[...] + 1.0)
        gates = jnp.dot(h_ref[...], whh_ref[...],
                        preferred_element_type=jnp.float32) + bihh_ref[...]
        bgv = bg_ref[...]
        for j in range(4):
            gj = jnp.maximum(
                dv * (acc_ref[j] + xs_ref[j]) + bgv[:, j * 128:(j + 1) * 128], 0.0)
            gates = gates + jnp.dot(gj, wih_ref[j * 128:(j + 1) * 128, :],
                                    preferred_element_type=jnp.float32)
        ig = jax.nn.sigmoid(gates[:, :H])
        fg = jax.nn.sigmoid(gates[:, H:2 * H])
        gg = jnp.tanh(gates[:, 2 * H:3 * H])
        og = jax.nn.sigmoid(gates[:, 3 * H:])
        cn = fg * c_ref[...] + ig * gg
        hn = og * jnp.tanh(cn)
        cn_ref[...] = cn
        hn_ref[...] = hn
        out_ref[...] = jnp.dot(hn, wo_ref[...],
                               preferred_element_type=jnp.float32) + bo_ref[...]

    return pl.pallas_call(
        lstm_kernel,
        grid=(nb,),
        in_specs=[
            pl.BlockSpec((4, rb, 128), lambda i: (0, i, 0)),
            pl.BlockSpec((4, rb, 128), lambda i: (0, i, 0)),
            pl.BlockSpec((rb, 1), lambda i: (i, 0)),
            pl.BlockSpec((rb, H), lambda i: (i, 0)),
            pl.BlockSpec((rb, H), lambda i: (i, 0)),
            pl.BlockSpec((H, 4 * H), lambda i: (0, 0)),
            pl.BlockSpec((512, 4 * H), lambda i: (0, 0)),
            pl.BlockSpec((1, 4 * H), lambda i: (0, 0)),
            pl.BlockSpec((1, 512), lambda i: (0, 0)),
            pl.BlockSpec((H, wo.shape[1]), lambda i: (0, 0)),
            pl.BlockSpec((1, wo.shape[1]), lambda i: (0, 0)),
        ],
        out_specs=[
            pl.BlockSpec((rb, wo.shape[1]), lambda i: (i, 0)),
            pl.BlockSpec((rb, H), lambda i: (i, 0)),
            pl.BlockSpec((rb, H), lambda i: (i, 0)),
        ],
        out_shape=[
            jax.ShapeDtypeStruct((N_NODES, wo.shape[1]), jnp.float32),
            jax.ShapeDtypeStruct((N_NODES, H), jnp.float32),
            jax.ShapeDtypeStruct((N_NODES, H), jnp.float32),
        ],
    )(acc4, xs4, dv2, h, c, whhT, wihT, bihh, bg, wo, bo)


def kernel(x, edge_index, h, c, W_gcn, b_gcn, W_ih, W_hh, b_ih, b_hh, W_out, b_out):
    src = edge_index[0]
    dst = edge_index[1]
    pad_e = EPAD - N_EDGES
    srcp = jnp.concatenate(
        [src, jnp.zeros((pad_e,), jnp.int32)]).reshape(NS, TB, EPB)
    dstp = jnp.concatenate(
        [dst, jnp.full((pad_e,), NPAD - 1, jnp.int32)]).reshape(NS, TB, EPB)
    sadj = (jnp.arange(4, dtype=jnp.int32) * N_NODES)[:, None, None, None] + srcp[None]
    trash = (NHALF + (jnp.arange(EPAD, dtype=jnp.int32) % 64)).reshape(NS, TB, EPB)
    dfl = dstp
    dst2 = jnp.stack([
        jnp.where(dfl < NHALF, dfl, trash),
        jnp.where(dfl >= NHALF, dfl - NHALF, trash),
    ])

    degp = _sc_deg(dstp.reshape(NC * NS, TB // 2, EPB))
    dv2 = (degp[:NPAD] + degp[NPAD:])[:N_NODES].reshape(N_NODES, 1)
    xs4 = _tc_gcn_mm(x, dv2, W_gcn)
    zz = jnp.zeros((HROWS, 128), jnp.float32)
    acc_flat = _sc_scatter(xs4.reshape(4 * N_NODES, 128), sadj, dst2, zz)
    acc4 = acc_flat.reshape(4, NPAD, 128)

    out, hn, cn = _tc_lstm(
        acc4, xs4, dv2, h, c, W_hh.T, W_ih.T,
        (b_ih + b_hh).reshape(1, -1), b_gcn.reshape(1, -1),
        W_out, b_out.reshape(1, -1))
    return (out, hn, cn)


# speedup vs baseline: 4.0084x; 1.0905x over previous
import jax
import jax.numpy as jnp
from jax import lax
from jax.experimental import pallas as pl
from jax.experimental.pallas import tpu as pltpu
from jax.experimental.pallas import tpu_sc as plsc

N_NODES = 10000
N_EDGES = 160000
NPAD = 10240
EPB = 128
TB = 80
EPAD = 16 * TB * EPB
NC = 2
NS = 16
ROWS_PER_TILE = NPAD // NS

def _mesh():
    return plsc.VectorSubcoreMesh(
        core_axis_name="c", subcore_axis_name="s", num_cores=NC, num_subcores=NS)


def _sc_deg(dstp32):
    @pl.kernel(
        out_type=jax.ShapeDtypeStruct((NC * NPAD,), jnp.float32),
        mesh=_mesh(),
        scratch_types=[
            pltpu.VMEM((TB // 2, EPB), jnp.int32),
            pltpu.VMEM((EPB,), jnp.float32),
            pltpu.VMEM((ROWS_PER_TILE,), jnp.float32),
            pltpu.VMEM_SHARED((NPAD,), jnp.float32),
        ],
    )
    def deg_kernel(dst_hbm, deg_hbm, dst_v, ones_v, z_v, deg_sp):
        c = lax.axis_index("c")
        s = lax.axis_index("s")
        gw = c * NS + s
        for i in range(EPB // 16):
            ones_v[pl.ds(i * 16, 16)] = jnp.full((16,), 1.0, jnp.float32)
        for i in range(ROWS_PER_TILE // 16):
            z_v[pl.ds(i * 16, 16)] = jnp.zeros((16,), jnp.float32)
        pltpu.sync_copy(z_v, deg_sp.at[pl.ds(s * ROWS_PER_TILE, ROWS_PER_TILE)])
        pltpu.sync_copy(dst_hbm.at[gw], dst_v)
        plsc.subcore_barrier()
        def blk(b, carry):
            pltpu.sync_copy(ones_v, deg_sp.at[dst_v.at[b]], add=True)
            return carry
        lax.fori_loop(0, TB // 2, blk, 0)
        plsc.subcore_barrier()
        pltpu.sync_copy(deg_sp.at[pl.ds(s * ROWS_PER_TILE, ROWS_PER_TILE)],
                        deg_hbm.at[pl.ds(c * NPAD + s * ROWS_PER_TILE, ROWS_PER_TILE)])

    return deg_kernel(dstp32)


def _tc_gcn_mm(x, dv2, w):
    rb = 400
    nb = N_NODES // rb

    def mm_kernel(x_ref, deg_ref, w_ref, o_ref):
        dv = lax.rsqrt(deg_ref[...] + 1.0)
        o_ref[0] = jnp.dot(x_ref[...] * dv, w_ref[...],
                           preferred_element_type=jnp.float32)

    return pl.pallas_call(
        mm_kernel,
        grid=(nb, 4),
        in_specs=[
            pl.BlockSpec((rb, x.shape[1]), lambda i, j: (i, 0)),
            pl.BlockSpec((rb, 1), lambda i, j: (i, 0)),
            pl.BlockSpec((x.shape[1], 128), lambda i, j: (0, j)),
        ],
        out_specs=pl.BlockSpec((1, rb, 128), lambda i, j: (j, i, 0)),
        out_shape=jax.ShapeDtypeStruct((4, N_NODES, 128), jnp.float32),
    )(x, dv2, w)


NHALF = NPAD // 2
ACC_ROWS = NHALF + 64
HROWS = NHALF // NS


def _sc_scatter(xs4, srcp, dst2, zz):
    @pl.kernel(
        out_type=jax.ShapeDtypeStruct((8 * NHALF, 128), jnp.float32),
        mesh=_mesh(),
        scratch_types=[
            pltpu.VMEM((TB, EPB), jnp.int32),
            pltpu.VMEM((TB, EPB), jnp.int32),
            pltpu.VMEM((2, EPB, 128), jnp.float32),
            pltpu.VMEM((HROWS // 4, 128), jnp.float32),
            pltpu.VMEM_SHARED((ACC_ROWS, 128), jnp.float32),
            pltpu.SemaphoreType.DMA,
            pltpu.SemaphoreType.DMA,
        ],
    )
    def msg_kernel(xs_hbm, src_hbm, dst_hbm, zz_hbm, acc_hbm,
                   idx_v, dst_v, g_v, z_v, acc_sp, sem_g, sem_s):
        c = lax.axis_index("c")
        s = lax.axis_index("s")
        pltpu.sync_copy(zz_hbm, z_v)
        for t in range(4):
            chunk = c + 2 * (t // 2)
            half = t % 2
            for zi in range(4):
                pltpu.sync_copy(
                    z_v, acc_sp.at[pl.ds(s * HROWS + zi * (HROWS // 4), HROWS // 4)])
            pltpu.sync_copy(src_hbm.at[s], idx_v)
            pltpu.sync_copy(dst_hbm.at[half, s], dst_v)
            plsc.subcore_barrier()
            def grp(gi, carry):
                gd = []
                for i in range(2):
                    b = gi * 2 + i
                    gd.append(pltpu.async_copy(
                        xs_hbm.at[chunk].at[idx_v.at[b]], g_v.at[i], sem_g))
                sd = []
                for i in range(2):
                    b = gi * 2 + i
                    gd[i].wait()
                    sd.append(pltpu.async_copy(
                        g_v.at[i], acc_sp.at[dst_v.at[b]], sem_s, add=True))
                for d in sd:
                    d.wait()
                return carry
            lax.fori_loop(0, TB // 2, grp, 0)
            plsc.subcore_barrier()
            pltpu.sync_copy(
                acc_sp.at[pl.ds(s * HROWS, HROWS)],
                acc_hbm.at[pl.ds((chunk * 2 + half) * NHALF + s * HROWS, HROWS)])
            plsc.subcore_barrier()

    return msg_kernel(xs4, srcp, dst2, zz)


def _tc_lstm(acc4, xs4, dv2, h, c, whhT, wihT, bihh, bg, wo, bo):
    rb = 1000
    nb = N_NODES // rb
    H = h.shape[1]

    def lstm_kernel(acc_ref, xs_ref, deg_ref, h_ref, c_ref, whh_ref, wih_ref,
                    bihh_ref, bg_ref, wo_ref, bo_ref, out_ref, hn_ref, cn_ref):
        dv = lax.rsqrt(deg_ref[...] + 1.0)
        gates = jnp.dot(h_ref[...], whh_ref[...],
                        preferred_element_type=jnp.float32) + bihh_ref[...]
        bgv = bg_ref[...]
        for j in range(4):
            gj = jnp.maximum(
                dv * (acc_ref[j] + xs_ref[j]) + bgv[:, j * 128:(j + 1) * 128], 0.0)
            gates = gates + jnp.dot(gj, wih_ref[j * 128:(j + 1) * 128, :],
                                    preferred_element_type=jnp.float32)
        ig = jax.nn.sigmoid(gates[:, :H])
        fg = jax.nn.sigmoid(gates[:, H:2 * H])
        gg = jnp.tanh(gates[:, 2 * H:3 * H])
        og = jax.nn.sigmoid(gates[:, 3 * H:])
        cn = fg * c_ref[...] + ig * gg
        hn = og * jnp.tanh(cn)
        cn_ref[...] = cn
        hn_ref[...] = hn
        out_ref[...] = jnp.dot(hn, wo_ref[...],
                               preferred_element_type=jnp.float32) + bo_ref[...]

    return pl.pallas_call(
        lstm_kernel,
        grid=(nb,),
        in_specs=[
            pl.BlockSpec((4, rb, 128), lambda i: (0, i, 0)),
            pl.BlockSpec((4, rb, 128), lambda i: (0, i, 0)),
            pl.BlockSpec((rb, 1), lambda i: (i, 0)),
            pl.BlockSpec((rb, H), lambda i: (i, 0)),
            pl.BlockSpec((rb, H), lambda i: (i, 0)),
            pl.BlockSpec((H, 4 * H), lambda i: (0, 0)),
            pl.BlockSpec((512, 4 * H), lambda i: (0, 0)),
            pl.BlockSpec((1, 4 * H), lambda i: (0, 0)),
            pl.BlockSpec((1, 512), lambda i: (0, 0)),
            pl.BlockSpec((H, wo.shape[1]), lambda i: (0, 0)),
            pl.BlockSpec((1, wo.shape[1]), lambda i: (0, 0)),
        ],
        out_specs=[
            pl.BlockSpec((rb, wo.shape[1]), lambda i: (i, 0)),
            pl.BlockSpec((rb, H), lambda i: (i, 0)),
            pl.BlockSpec((rb, H), lambda i: (i, 0)),
        ],
        out_shape=[
            jax.ShapeDtypeStruct((N_NODES, wo.shape[1]), jnp.float32),
            jax.ShapeDtypeStruct((N_NODES, H), jnp.float32),
            jax.ShapeDtypeStruct((N_NODES, H), jnp.float32),
        ],
    )(acc4, xs4, dv2, h, c, whhT, wihT, bihh, bg, wo, bo)


def kernel(x, edge_index, h, c, W_gcn, b_gcn, W_ih, W_hh, b_ih, b_hh, W_out, b_out):
    src = edge_index[0]
    dst = edge_index[1]
    pad_e = EPAD - N_EDGES
    srcp = jnp.concatenate(
        [src, jnp.zeros((pad_e,), jnp.int32)]).reshape(NS, TB, EPB)
    dstp = jnp.concatenate(
        [dst, jnp.full((pad_e,), NPAD - 1, jnp.int32)]).reshape(NS, TB, EPB)
    trash = (NHALF + (jnp.arange(EPAD, dtype=jnp.int32) % 64)).reshape(NS, TB, EPB)
    dfl = dstp
    dst2 = jnp.stack([
        jnp.where(dfl < NHALF, dfl, trash),
        jnp.where(dfl >= NHALF, dfl - NHALF, trash),
    ])

    degp = _sc_deg(dstp.reshape(NC * NS, TB // 2, EPB))
    dv2 = (degp[:NPAD] + degp[NPAD:])[:N_NODES].reshape(N_NODES, 1)
    xs4 = _tc_gcn_mm(x, dv2, W_gcn)
    zz = jnp.zeros((HROWS // 4, 128), jnp.float32)
    acc_flat = _sc_scatter(xs4, srcp, dst2, zz)
    acc4 = acc_flat.reshape(4, NPAD, 128)

    out, hn, cn = _tc_lstm(
        acc4, xs4, dv2, h, c, W_hh.T, W_ih.T,
        (b_ih + b_hh).reshape(1, -1), b_gcn.reshape(1, -1),
        W_out, b_out.reshape(1, -1))
    return (out, hn, cn)


# speedup vs baseline: 4.1950x; 1.0466x over previous
import jax
import jax.numpy as jnp
from jax import lax
from jax.experimental import pallas as pl
from jax.experimental.pallas import tpu as pltpu
from jax.experimental.pallas import tpu_sc as plsc

N_NODES = 10000
N_EDGES = 160000
NPAD = 10240
EPB = 128
TB = 80
EPAD = 16 * TB * EPB
NC = 2
NS = 16
ROWS_PER_TILE = NPAD // NS

def _mesh():
    return plsc.VectorSubcoreMesh(
        core_axis_name="c", subcore_axis_name="s", num_cores=NC, num_subcores=NS)


def _sc_deg(dstp32):
    @pl.kernel(
        out_type=jax.ShapeDtypeStruct((NC * NPAD,), jnp.float32),
        mesh=_mesh(),
        scratch_types=[
            pltpu.VMEM((TB // 2, EPB), jnp.int32),
            pltpu.VMEM((EPB,), jnp.float32),
            pltpu.VMEM((ROWS_PER_TILE,), jnp.float32),
            pltpu.VMEM_SHARED((NPAD,), jnp.float32),
        ],
    )
    def deg_kernel(dst_hbm, deg_hbm, dst_v, ones_v, z_v, deg_sp):
        c = lax.axis_index("c")
        s = lax.axis_index("s")
        gw = c * NS + s
        for i in range(EPB // 16):
            ones_v[pl.ds(i * 16, 16)] = jnp.full((16,), 1.0, jnp.float32)
        for i in range(ROWS_PER_TILE // 16):
            z_v[pl.ds(i * 16, 16)] = jnp.zeros((16,), jnp.float32)
        pltpu.sync_copy(z_v, deg_sp.at[pl.ds(s * ROWS_PER_TILE, ROWS_PER_TILE)])
        pltpu.sync_copy(dst_hbm.at[gw], dst_v)
        plsc.subcore_barrier()
        def blk(b, carry):
            pltpu.sync_copy(ones_v, deg_sp.at[dst_v.at[b]], add=True)
            return carry
        lax.fori_loop(0, TB // 2, blk, 0)
        plsc.subcore_barrier()
        pltpu.sync_copy(deg_sp.at[pl.ds(s * ROWS_PER_TILE, ROWS_PER_TILE)],
                        deg_hbm.at[pl.ds(c * NPAD + s * ROWS_PER_TILE, ROWS_PER_TILE)])

    return deg_kernel(dstp32)


def _tc_gcn_mm(x, dv2, w):
    rb = 400
    nb = N_NODES // rb

    def mm_kernel(x_ref, deg_ref, w_ref, o_ref):
        dv = lax.rsqrt(deg_ref[...] + 1.0)
        o_ref[0] = jnp.dot(x_ref[...] * dv, w_ref[...],
                           preferred_element_type=jnp.float32)

    return pl.pallas_call(
        mm_kernel,
        grid=(nb, 4),
        in_specs=[
            pl.BlockSpec((rb, x.shape[1]), lambda i, j: (i, 0)),
            pl.BlockSpec((rb, 1), lambda i, j: (i, 0)),
            pl.BlockSpec((x.shape[1], 128), lambda i, j: (0, j)),
        ],
        out_specs=pl.BlockSpec((1, rb, 128), lambda i, j: (j, i, 0)),
        out_shape=jax.ShapeDtypeStruct((4, N_NODES, 128), jnp.float32),
    )(x, dv2, w)


NHALF = NPAD // 2
ACC_ROWS = NHALF + 32
HROWS = NHALF // NS


def _sc_scatter(xs4, srcp, dst2, zz):
    @pl.kernel(
        out_type=jax.ShapeDtypeStruct((8 * NHALF, 128), jnp.float32),
        mesh=_mesh(),
        scratch_types=[
            pltpu.VMEM((TB, EPB), jnp.int32),
            pltpu.VMEM((TB, EPB), jnp.int32),
            pltpu.VMEM((4, EPB, 128), jnp.float32),
            pltpu.VMEM_SHARED((ACC_ROWS, 128), jnp.float32),
            pltpu.SemaphoreType.DMA((4,)),
            pltpu.SemaphoreType.DMA((4,)),
        ],
    )
    def msg_kernel(xs_hbm, src_hbm, dst_hbm, zz_hbm, acc_hbm,
                   idx_v, dst_v, g_v, acc_sp, sem_g, sem_s):
        c = lax.axis_index("c")
        s = lax.axis_index("s")
        pltpu.sync_copy(src_hbm.at[s], idx_v)
        for t in range(4):
            chunk = c + 2 * (t // 2)
            half = t % 2
            tbl = xs_hbm.at[chunk]
            pltpu.sync_copy(zz_hbm, acc_sp.at[pl.ds(s * HROWS, HROWS)])
            pltpu.sync_copy(dst_hbm.at[half, s], dst_v)
            plsc.subcore_barrier()

            def gather(b, slot):
                return pltpu.make_async_copy(
                    tbl.at[idx_v.at[b]], g_v.at[slot], sem_g.at[slot])

            def scatter(b, slot):
                return pltpu.make_async_copy(
                    g_v.at[slot], acc_sp.at[dst_v.at[b]], sem_s.at[slot])

            def pipe(b, wait_s, issue_g):
                m = b % 4
                gather(b, m).wait()
                scatter(b, m).start(add=True)
                k = (b + 2) % 4
                if wait_s:
                    scatter(b - 2, k).wait()
                if issue_g:
                    gather(b + 2, k).start()

            gather(0, 0).start()
            gather(1, 1).start()
            pipe(0, False, True)
            pipe(1, False, True)
            def body(b, carry):
                pipe(b, True, True)
                return carry
            lax.fori_loop(2, TB - 2, body, 0)
            pipe(TB - 2, True, False)
            pipe(TB - 1, True, False)
            scatter(TB - 2, (TB - 2) % 4).wait()
            scatter(TB - 1, (TB - 1) % 4).wait()
            plsc.subcore_barrier()
            pltpu.sync_copy(
                acc_sp.at[pl.ds(s * HROWS, HROWS)],
                acc_hbm.at[pl.ds((chunk * 2 + half) * NHALF + s * HROWS, HROWS)])
            plsc.subcore_barrier()

    return msg_kernel(xs4, srcp, dst2, zz)


def _tc_lstm(acc4, xs4, dv2, h, c, whhT, wihT, bihh, bg, wo, bo):
    rb = 1000
    nb = N_NODES // rb
    H = h.shape[1]

    def lstm_kernel(acc_ref, xs_ref, deg_ref, h_ref, c_ref, whh_ref, wih_ref,
                    bihh_ref, bg_ref, wo_ref, bo_ref, out_ref, hn_ref, cn_ref):
        dv = lax.rsqrt(deg_ref[...] + 1.0)
        gates = jnp.dot(h_ref[...], whh_ref[...],
                        preferred_element_type=jnp.float32) + bihh_ref[...]
        bgv = bg_ref[...]
        for j in range(4):
            gj = jnp.maximum(
                dv * (acc_ref[j] + xs_ref[j]) + bgv[:, j * 128:(j + 1) * 128], 0.0)
            gates = gates + jnp.dot(gj, wih_ref[j * 128:(j + 1) * 128, :],
                                    preferred_element_type=jnp.float32)
        ig = jax.nn.sigmoid(gates[:, :H])
        fg = jax.nn.sigmoid(gates[:, H:2 * H])
        gg = jnp.tanh(gates[:, 2 * H:3 * H])
        og = jax.nn.sigmoid(gates[:, 3 * H:])
        cn = fg * c_ref[...] + ig * gg
        hn = og * jnp.tanh(cn)
        cn_ref[...] = cn
        hn_ref[...] = hn
        out_ref[...] = jnp.dot(hn, wo_ref[...],
                               preferred_element_type=jnp.float32) + bo_ref[...]

    return pl.pallas_call(
        lstm_kernel,
        grid=(nb,),
        in_specs=[
            pl.BlockSpec((4, rb, 128), lambda i: (0, i, 0)),
            pl.BlockSpec((4, rb, 128), lambda i: (0, i, 0)),
            pl.BlockSpec((rb, 1), lambda i: (i, 0)),
            pl.BlockSpec((rb, H), lambda i: (i, 0)),
            pl.BlockSpec((rb, H), lambda i: (i, 0)),
            pl.BlockSpec((H, 4 * H), lambda i: (0, 0)),
            pl.BlockSpec((512, 4 * H), lambda i: (0, 0)),
            pl.BlockSpec((1, 4 * H), lambda i: (0, 0)),
            pl.BlockSpec((1, 512), lambda i: (0, 0)),
            pl.BlockSpec((H, wo.shape[1]), lambda i: (0, 0)),
            pl.BlockSpec((1, wo.shape[1]), lambda i: (0, 0)),
        ],
        out_specs=[
            pl.BlockSpec((rb, wo.shape[1]), lambda i: (i, 0)),
            pl.BlockSpec((rb, H), lambda i: (i, 0)),
            pl.BlockSpec((rb, H), lambda i: (i, 0)),
        ],
        out_shape=[
            jax.ShapeDtypeStruct((N_NODES, wo.shape[1]), jnp.float32),
            jax.ShapeDtypeStruct((N_NODES, H), jnp.float32),
            jax.ShapeDtypeStruct((N_NODES, H), jnp.float32),
        ],
    )(acc4, xs4, dv2, h, c, whhT, wihT, bihh, bg, wo, bo)


def kernel(x, edge_index, h, c, W_gcn, b_gcn, W_ih, W_hh, b_ih, b_hh, W_out, b_out):
    src = edge_index[0]
    dst = edge_index[1]
    pad_e = EPAD - N_EDGES
    srcp = jnp.concatenate(
        [src, jnp.zeros((pad_e,), jnp.int32)]).reshape(NS, TB, EPB)
    dstp = jnp.concatenate(
        [dst, jnp.full((pad_e,), NPAD - 1, jnp.int32)]).reshape(NS, TB, EPB)
    trash = (NHALF + (jnp.arange(EPAD, dtype=jnp.int32) % 32)).reshape(NS, TB, EPB)
    dfl = dstp
    dst2 = jnp.stack([
        jnp.where(dfl < NHALF, dfl, trash),
        jnp.where(dfl >= NHALF, dfl - NHALF, trash),
    ])

    degp = _sc_deg(dstp.reshape(NC * NS, TB // 2, EPB))
    dv2 = (degp[:NPAD] + degp[NPAD:])[:N_NODES].reshape(N_NODES, 1)
    xs4 = _tc_gcn_mm(x, dv2, W_gcn)
    zz = jnp.zeros((HROWS, 128), jnp.float32)
    acc_flat = _sc_scatter(xs4, srcp, dst2, zz)
    acc4 = acc_flat.reshape(4, NPAD, 128)

    out, hn, cn = _tc_lstm(
        acc4, xs4, dv2, h, c, W_hh.T, W_ih.T,
        (b_ih + b_hh).reshape(1, -1), b_gcn.reshape(1, -1),
        W_out, b_out.reshape(1, -1))
    return (out, hn, cn)


# speedup vs baseline: 7.7885x; 1.8566x over previous
import jax
import jax.numpy as jnp
from jax import lax
from jax.experimental import pallas as pl
from jax.experimental.pallas import tpu as pltpu
from jax.experimental.pallas import tpu_sc as plsc

N_NODES = 10000
N_EDGES = 160000
NPAD = 10240
EPB = 128
TB = 80
EPAD = 16 * TB * EPB
NC = 2
NS = 16
ROWS_PER_TILE = NPAD // NS

ACC2 = 10112
HR2 = ACC2 // NS
SUB = 40


def _mesh():
    return plsc.VectorSubcoreMesh(
        core_axis_name="c", subcore_axis_name="s", num_cores=NC, num_subcores=NS)


def _sc_deg(dstp32):
    @pl.kernel(
        out_type=jax.ShapeDtypeStruct((NC * NPAD,), jnp.float32),
        mesh=_mesh(),
        scratch_types=[
            pltpu.VMEM((TB // 2, EPB), jnp.int32),
            pltpu.VMEM((EPB,), jnp.float32),
            pltpu.VMEM((ROWS_PER_TILE,), jnp.float32),
            pltpu.VMEM_SHARED((NPAD,), jnp.float32),
        ],
    )
    def deg_kernel(dst_hbm, deg_hbm, dst_v, ones_v, z_v, deg_sp):
        c = lax.axis_index("c")
        s = lax.axis_index("s")
        gw = c * NS + s
        for i in range(EPB // 16):
            ones_v[pl.ds(i * 16, 16)] = jnp.full((16,), 1.0, jnp.float32)
        for i in range(ROWS_PER_TILE // 16):
            z_v[pl.ds(i * 16, 16)] = jnp.zeros((16,), jnp.float32)
        pltpu.sync_copy(z_v, deg_sp.at[pl.ds(s * ROWS_PER_TILE, ROWS_PER_TILE)])
        pltpu.sync_copy(dst_hbm.at[gw], dst_v)
        plsc.subcore_barrier()
        def blk(b, carry):
            pltpu.sync_copy(ones_v, deg_sp.at[dst_v.at[b]], add=True)
            return carry
        lax.fori_loop(0, TB // 2, blk, 0)
        plsc.subcore_barrier()
        pltpu.sync_copy(deg_sp.at[pl.ds(s * ROWS_PER_TILE, ROWS_PER_TILE)],
                        deg_hbm.at[pl.ds(c * NPAD + s * ROWS_PER_TILE, ROWS_PER_TILE)])

    return deg_kernel(dstp32)


def _tc_gcn_mm(x, dv2, w):
    rb = 400
    nb = N_NODES // rb

    def mm_kernel(x_ref, deg_ref, w_ref, o_ref):
        dv = lax.rsqrt(deg_ref[...] + 1.0)
        o_ref[0] = jnp.dot(x_ref[...] * dv, w_ref[...],
                           preferred_element_type=jnp.float32)

    return pl.pallas_call(
        mm_kernel,
        grid=(nb, 4),
        in_specs=[
            pl.BlockSpec((rb, x.shape[1]), lambda i, j: (i, 0)),
            pl.BlockSpec((rb, 1), lambda i, j: (i, 0)),
            pl.BlockSpec((x.shape[1], 128), lambda i, j: (0, j)),
        ],
        out_specs=pl.BlockSpec((1, rb, 128), lambda i, j: (j, i, 0)),
        out_shape=jax.ShapeDtypeStruct((4, N_NODES, 128), jnp.float32),
    )(x, dv2, w)


def _sc_scatter(xs4, ed, zz):
    @pl.kernel(
        out_type=jax.ShapeDtypeStruct((4 * ACC2, 128), jnp.float32),
        mesh=_mesh(),
        scratch_types=[
            pltpu.VMEM((2, SUB, EPB), jnp.int32),
            pltpu.VMEM((2, EPB, 128), jnp.float32),
            pltpu.VMEM_SHARED((ACC2, 128), jnp.float32),
            pltpu.SemaphoreType.DMA((2,)),
            pltpu.SemaphoreType.DMA((2,)),
        ],
    )
    def msg_kernel(xs_hbm, ed_hbm, zz_hbm, acc_hbm, ed_v, g_v, acc_sp,
                   sem_g, sem_s):
        c = lax.axis_index("c")
        s = lax.axis_index("s")
        idx_v = ed_v.at[0]
        dst_v = ed_v.at[1]
        for t in range(2):
            chunk = c + 2 * t
            tbl = xs_hbm.at[chunk]
            pltpu.sync_copy(zz_hbm, acc_sp.at[pl.ds(s * HR2, HR2)])
            plsc.subcore_barrier()

            def gather(b, slot):
                return pltpu.make_async_copy(
                    tbl.at[idx_v.at[b]], g_v.at[slot], sem_g.at[slot])

            def scatter(b, slot):
                return pltpu.make_async_copy(
                    g_v.at[slot], acc_sp.at[dst_v.at[b]], sem_s.at[slot])

            def pipe(b, wait_s, issue_g):
                m = b % 2
                gather(b, m).wait()
                scatter(b, m).start(add=True)
                if wait_s:
                    scatter(b - 1, 1 - m).wait()
                if issue_g:
                    gather(b + 1, 1 - m).start()

            for sub in range(TB // SUB):
                pltpu.sync_copy(ed_hbm.at[2 * s + sub], ed_v)
                gather(0, 0).start()
                pipe(0, False, True)
                def body(b, carry):
                    pipe(b, True, True)
                    return carry
                lax.fori_loop(1, SUB - 1, body, 0)
                pipe(SUB - 1, True, False)
                scatter(SUB - 1, (SUB - 1) % 2).wait()
            plsc.subcore_barrier()
            pltpu.sync_copy(
                acc_sp.at[pl.ds(s * HR2, HR2)],
                acc_hbm.at[pl.ds(chunk * ACC2 + s * HR2, HR2)])
            plsc.subcore_barrier()

    return msg_kernel(xs4, ed, zz)


def _tc_lstm(acc4, xs4, dv2, h, c, whhT, wihT, bihh, bg, wo, bo):
    rb = 1000
    nb = N_NODES // rb
    H = h.shape[1]

    def lstm_kernel(acc_ref, xs_ref, deg_ref, h_ref, c_ref, whh_ref, wih_ref,
                    bihh_ref, bg_ref, wo_ref, bo_ref, out_ref, hn_ref, cn_ref):
        dv = lax.rsqrt(deg_ref[...] + 1.0)
        gates = jnp.dot(h_ref[...], whh_ref[...],
                        preferred_element_type=jnp.float32) + bihh_ref[...]
        bgv = bg_ref[...]
        for j in range(4):
            gj = jnp.maximum(
                dv * (acc_ref[j] + xs_ref[j]) + bgv[:, j * 128:(j + 1) * 128], 0.0)
            gates = gates + jnp.dot(gj, wih_ref[j * 128:(j + 1) * 128, :],
                                    preferred_element_type=jnp.float32)
        ig = jax.nn.sigmoid(gates[:, :H])
        fg = jax.nn.sigmoid(gates[:, H:2 * H])
        gg = jnp.tanh(gates[:, 2 * H:3 * H])
        og = jax.nn.sigmoid(gates[:, 3 * H:])
        cn = fg * c_ref[...] + ig * gg
        hn = og * jnp.tanh(cn)
        cn_ref[...] = cn
        hn_ref[...] = hn
        out_ref[...] = jnp.dot(hn, wo_ref[...],
                               preferred_element_type=jnp.float32) + bo_ref[...]

    return pl.pallas_call(
        lstm_kernel,
        grid=(nb,),
        in_specs=[
            pl.BlockSpec((4, rb, 128), lambda i: (0, i, 0)),
            pl.BlockSpec((4, rb, 128), lambda i: (0, i, 0)),
            pl.BlockSpec((rb, 1), lambda i: (i, 0)),
            pl.BlockSpec((rb, H), lambda i: (i, 0)),
            pl.BlockSpec((rb, H), lambda i: (i, 0)),
            pl.BlockSpec((H, 4 * H), lambda i: (0, 0)),
            pl.BlockSpec((512, 4 * H), lambda i: (0, 0)),
            pl.BlockSpec((1, 4 * H), lambda i: (0, 0)),
            pl.BlockSpec((1, 512), lambda i: (0, 0)),
            pl.BlockSpec((H, wo.shape[1]), lambda i: (0, 0)),
            pl.BlockSpec((1, wo.shape[1]), lambda i: (0, 0)),
        ],
        out_specs=[
            pl.BlockSpec((rb, wo.shape[1]), lambda i: (i, 0)),
            pl.BlockSpec((rb, H), lambda i: (i, 0)),
            pl.BlockSpec((rb, H), lambda i: (i, 0)),
        ],
        out_shape=[
            jax.ShapeDtypeStruct((N_NODES, wo.shape[1]), jnp.float32),
            jax.ShapeDtypeStruct((N_NODES, H), jnp.float32),
            jax.ShapeDtypeStruct((N_NODES, H), jnp.float32),
        ],
    )(acc4, xs4, dv2, h, c, whhT, wihT, bihh, bg, wo, bo)


def kernel(x, edge_index, h, c, W_gcn, b_gcn, W_ih, W_hh, b_ih, b_hh, W_out, b_out):
    src = edge_index[0]
    dst = edge_index[1]
    pad_e = EPAD - N_EDGES
    pad_dst = N_NODES + (jnp.arange(pad_e, dtype=jnp.int32) % (ACC2 - N_NODES))
    srcp = jnp.concatenate([src, jnp.zeros((pad_e,), jnp.int32)])
    dstp = jnp.concatenate([dst, pad_dst])

    degp = _sc_deg(dstp.reshape(NC * NS, TB // 2, EPB))
    dv2 = (degp[:NPAD] + degp[NPAD:])[:N_NODES].reshape(N_NODES, 1)
    xs4 = _tc_gcn_mm(x, dv2, W_gcn)

    ed = jnp.stack([srcp.reshape(NS * (TB // SUB), SUB, EPB),
                    dstp.reshape(NS * (TB // SUB), SUB, EPB)], axis=1)
    zz = jnp.zeros((HR2, 128), jnp.float32)
    acc_flat = _sc_scatter(xs4, ed, zz)
    acc4 = acc_flat.reshape(4, ACC2, 128)

    out, hn, cn = _tc_lstm(
        acc4, xs4, dv2, h, c, W_hh.T, W_ih.T,
        (b_ih + b_hh).reshape(1, -1), b_gcn.reshape(1, -1),
        W_out, b_out.reshape(1, -1))
    return (out, hn, cn)
